# baseline (speedup 1.0000x reference)
import functools

import jax
import jax.numpy as jnp
from jax import lax
from jax.experimental import pallas as pl
from jax.experimental.pallas import tpu as pltpu
from jax.experimental.pallas import tpu_sc as plsc

NC = 2
NS = 16
NW = NC * NS
LANES = 16

BATCH = 16384
EMBED_DIM = 64
B_PER_W = BATCH // NW
CHUNK = 128
N_CHUNKS = B_PER_W // CHUNK
HALF = B_PER_W // 2
ROW_W = 2 * EMBED_DIM


def _mf_body(urow_ref, irow_ref, uorig_ref, iorig_ref, upar_ref, ipar_ref,
             uemb_ref, iemb_ref, gb_ref, ubias_ref, ibias_ref, out_ref,
             idx_u, idx_i, idxo_u, idxo_i, u_rows, i_rows, pu_v, pi_v, ub_v,
             ib_v, gb_v, out_v, sem, bsem):
    wid = lax.axis_index("s") * NC + lax.axis_index("c")

    pltpu.sync_copy(urow_ref.at[wid], idx_u)
    pltpu.sync_copy(irow_ref.at[wid], idx_i)
    pltpu.sync_copy(uorig_ref.at[wid], idxo_u)
    pltpu.sync_copy(iorig_ref.at[wid], idxo_i)
    pltpu.sync_copy(upar_ref.at[wid], pu_v)
    pltpu.sync_copy(ipar_ref.at[wid], pi_v)
    pltpu.sync_copy(gb_ref, gb_v)

    bias_copies = []
    for c in range(N_CHUNKS):
        sl = pl.ds(c * CHUNK, CHUNK)
        bias_copies.append(
            pltpu.async_copy(ubias_ref.at[idxo_u.at[c]], ub_v.at[sl], bsem))
        bias_copies.append(
            pltpu.async_copy(ibias_ref.at[idxo_i.at[c]], ib_v.at[sl], bsem))

    lane = lax.iota(jnp.int32, LANES)

    def gather_half(h):
        copies = []
        for c in range(2):
            g = 2 * h + c
            sl = pl.ds(c * CHUNK, CHUNK)
            copies.append(
                pltpu.async_copy(uemb_ref.at[idx_u.at[g]], u_rows.at[sl], sem))
            copies.append(
                pltpu.async_copy(iemb_ref.at[idx_i.at[g]], i_rows.at[sl], sem))
        return copies

    def compute_half(h, gb_vec):
        hbase = h * HALF

        def group_body(g, _):
            lb = g * LANES
            base = hbase + lb
            row = lb + lane
            ucol0 = pu_v[pl.ds(base, LANES)] * EMBED_DIM
            icol0 = pi_v[pl.ds(base, LANES)] * EMBED_DIM

            def dot_step(j, acc):
                col = (j & 48) + ((lane + j) & 15)
                u = plsc.load_gather(u_rows, [row, ucol0 + col])
                v = plsc.load_gather(i_rows, [row, icol0 + col])
                return acc + u * v

            acc0 = gb_vec + ub_v[pl.ds(base, LANES)] + ib_v[pl.ds(base, LANES)]
            acc = lax.fori_loop(0, EMBED_DIM, dot_step, acc0)
            out_v[pl.ds(base, LANES)] = acc
            return 0

        lax.fori_loop(0, HALF // LANES, group_body, 0)

    cps = gather_half(0)
    for cp in bias_copies:
        cp.wait()
    gb_vec = gb_v[...]
    for cp in cps:
        cp.wait()
    compute_half(0, gb_vec)
    cps = gather_half(1)
    for cp in cps:
        cp.wait()
    compute_half(1, gb_vec)

    pltpu.sync_copy(out_v, out_ref.at[wid])


@jax.jit
def _mf_sc(urow3, irow3, uorig3, iorig3, upar2, ipar2, uemb2, iemb2, gb16,
           user_bias, item_bias):
    mesh = plsc.VectorSubcoreMesh(core_axis_name="c", subcore_axis_name="s")
    k = pl.kernel(
        _mf_body,
        out_type=jax.ShapeDtypeStruct((NW, B_PER_W), jnp.float32),
        mesh=mesh,
        compiler_params=pltpu.CompilerParams(needs_layout_passes=False),
        scratch_types=[
            pltpu.VMEM((N_CHUNKS, CHUNK), jnp.int32),
            pltpu.VMEM((N_CHUNKS, CHUNK), jnp.int32),
            pltpu.VMEM((N_CHUNKS, CHUNK), jnp.int32),
            pltpu.VMEM((N_CHUNKS, CHUNK), jnp.int32),
            pltpu.VMEM((HALF, ROW_W), jnp.float32),
            pltpu.VMEM((HALF, ROW_W), jnp.float32),
            pltpu.VMEM((B_PER_W,), jnp.int32),
            pltpu.VMEM((B_PER_W,), jnp.int32),
            pltpu.VMEM((B_PER_W,), jnp.float32),
            pltpu.VMEM((B_PER_W,), jnp.float32),
            pltpu.VMEM((LANES,), jnp.float32),
            pltpu.VMEM((B_PER_W,), jnp.float32),
            pltpu.SemaphoreType.DMA,
            pltpu.SemaphoreType.DMA,
        ],
    )
    return k(urow3, irow3, uorig3, iorig3, upar2, ipar2, uemb2, iemb2, gb16,
             user_bias, item_bias)


def kernel(user, item, user_emb, item_emb, global_bias, user_bias, item_bias):
    urow3 = (user >> 1).reshape(NW, N_CHUNKS, CHUNK)
    irow3 = (item >> 1).reshape(NW, N_CHUNKS, CHUNK)
    upar2 = (user & 1).reshape(NW, B_PER_W)
    ipar2 = (item & 1).reshape(NW, B_PER_W)
    uemb2 = user_emb.reshape(-1, ROW_W)
    iemb2 = item_emb.reshape(-1, ROW_W)
    gb16 = jnp.broadcast_to(global_bias, (LANES,))
    uorig3 = user.reshape(NW, N_CHUNKS, CHUNK)
    iorig3 = item.reshape(NW, N_CHUNKS, CHUNK)
    out = _mf_sc(urow3, irow3, uorig3, iorig3, upar2, ipar2, uemb2, iemb2,
                 gb16, user_bias, item_bias)
    return out.reshape(BATCH)


# speedup vs baseline: 2.1928x; 2.1928x over previous
import jax
import jax.numpy as jnp
from jax import lax
from jax.experimental import pallas as pl
from jax.experimental.pallas import tpu as pltpu
from jax.experimental.pallas import tpu_sc as plsc

NC = 2
NS = 16
NW = NC * NS
LANES = 16

BATCH = 16384
EMBED_DIM = 64
SLAB = 8
B_PER_W = BATCH // NW
CHUNK = 128
N_CHUNKS = B_PER_W // CHUNK
RCH = 32
NRND = B_PER_W // RCH


def _mf_body(uorig_ref, iorig_ref, uemb_ref, iemb_ref, gb_ref, ubias_ref,
             ibias_ref, out_ref,
             idxo_u, idxo_i, u_slabs, i_slabs, ub_v, ib_v, gb_v, out_v,
             sem, bsem):
    wid = lax.axis_index("s") * NC + lax.axis_index("c")

    pltpu.sync_copy(uorig_ref.at[wid], idxo_u)
    pltpu.sync_copy(iorig_ref.at[wid], idxo_i)
    pltpu.sync_copy(gb_ref, gb_v)

    bias_copies = []
    for c in range(N_CHUNKS):
        sl = pl.ds(c * CHUNK, CHUNK)
        bias_copies.append(
            pltpu.async_copy(ubias_ref.at[idxo_u.at[c]], ub_v.at[sl], bsem))
        bias_copies.append(
            pltpu.async_copy(ibias_ref.at[idxo_i.at[c]], ib_v.at[sl], bsem))
    for cp in bias_copies:
        cp.wait()

    lane = lax.iota(jnp.int32, LANES)
    gb_vec = gb_v[...]

    def round_body(r, _):
        def enq_group(g, _):
            base = r * RCH + g * LANES
            iu = idxo_u[base >> 7, pl.ds(base & 127, LANES)]
            ii = idxo_i[base >> 7, pl.ds(base & 127, LANES)]
            su_vec = iu >> 3
            si_vec = ii >> 3
            for t in range(LANES):
                e_loc = g * LANES + t
                pltpu.async_copy(uemb_ref.at[su_vec[t]], u_slabs.at[e_loc],
                                 sem)
                pltpu.async_copy(iemb_ref.at[si_vec[t]], i_slabs.at[e_loc],
                                 sem)
            return 0

        lax.fori_loop(0, RCH // LANES, enq_group, 0)

        pltpu.make_async_copy(uemb_ref.at[pl.ds(0, RCH)], u_slabs, sem).wait()
        pltpu.make_async_copy(uemb_ref.at[pl.ds(0, RCH)], i_slabs, sem).wait()

        def group(g, _):
            base = r * RCH + g * LANES
            iu = idxo_u[base >> 7, pl.ds(base & 127, LANES)]
            ii = idxo_i[base >> 7, pl.ds(base & 127, LANES)]
            urow = iu & 7
            irow = ii & 7
            p_vec = g * LANES + lane

            def dot_step(j, acc):
                col = (j & 48) + ((lane + j) & 15)
                u = plsc.load_gather(u_slabs, [p_vec, urow, col])
                v = plsc.load_gather(i_slabs, [p_vec, irow, col])
                return acc + u * v

            acc0 = gb_vec + ub_v[pl.ds(base, LANES)] + ib_v[pl.ds(base, LANES)]
            acc = lax.fori_loop(0, EMBED_DIM, dot_step, acc0)
            out_v[pl.ds(base, LANES)] = acc
            return 0

        lax.fori_loop(0, RCH // LANES, group, 0)
        return 0

    lax.fori_loop(0, NRND, round_body, 0)

    pltpu.sync_copy(out_v, out_ref.at[wid])


@jax.jit
def _mf_sc(uorig3, iorig3, uemb3, iemb3, gb16, user_bias, item_bias):
    mesh = plsc.VectorSubcoreMesh(core_axis_name="c", subcore_axis_name="s")
    k = pl.kernel(
        _mf_body,
        out_type=jax.ShapeDtypeStruct((NW, B_PER_W), jnp.float32),
        mesh=mesh,
        compiler_params=pltpu.CompilerParams(needs_layout_passes=False),
        scratch_types=[
            pltpu.VMEM((N_CHUNKS, CHUNK), jnp.int32),
            pltpu.VMEM((N_CHUNKS, CHUNK), jnp.int32),
            pltpu.VMEM((RCH, SLAB, EMBED_DIM), jnp.float32),
            pltpu.VMEM((RCH, SLAB, EMBED_DIM), jnp.float32),
            pltpu.VMEM((B_PER_W,), jnp.float32),
            pltpu.VMEM((B_PER_W,), jnp.float32),
            pltpu.VMEM((LANES,), jnp.float32),
            pltpu.VMEM((B_PER_W,), jnp.float32),
            pltpu.SemaphoreType.DMA,
            pltpu.SemaphoreType.DMA,
        ],
    )
    return k(uorig3, iorig3, uemb3, iemb3, gb16, user_bias, item_bias)


def kernel(user, item, user_emb, item_emb, global_bias, user_bias, item_bias):
    uorig3 = user.reshape(NW, N_CHUNKS, CHUNK)
    iorig3 = item.reshape(NW, N_CHUNKS, CHUNK)
    uemb3 = user_emb.reshape(-1, SLAB, EMBED_DIM)
    iemb3 = item_emb.reshape(-1, SLAB, EMBED_DIM)
    gb16 = jnp.broadcast_to(global_bias, (LANES,))
    out = _mf_sc(uorig3, iorig3, uemb3, iemb3, gb16, user_bias, item_bias)
    return out.reshape(BATCH)
